# Initial kernel scaffold; baseline (speedup 1.0000x reference)
import jax
import jax.numpy as jnp
from jax.experimental import pallas as pl


def kernel(x, router_w, router_b, w1, w3, w2, sw1, sw3, sw2):
    raise NotImplementedError("write your pallas kernel here")


# speedup vs baseline: 1.1557x; 1.1557x over previous
import functools
import math

import jax
import jax.numpy as jnp
from jax import lax
from jax.experimental import pallas as pl
from jax.experimental.pallas import tpu as pltpu
from jax.experimental.pallas import tpu_sc as plsc

T, D, E, K, FF = 2048, 768, 8, 2, 1536
TM = 128
NP = T * K + E * TM
NB = NP // TM
EL = 128
NEG = -3e38
NW = 32


def _router_body(x_ref, rwt_ref, rb_ref, d0_ref, d1_ref, g0_ref, g1_ref,
                 cnt_ref):
    scale = 1.0 / math.sqrt(D)
    lanes = lax.broadcasted_iota(jnp.int32, (T, EL), 1)
    valid = lanes < E
    lg = jnp.dot(x_ref[:], rwt_ref[:], preferred_element_type=jnp.float32)
    lgv = jnp.where(valid, lg * scale, NEG)
    biased = lgv + rb_ref[:]
    m1 = jnp.max(biased, axis=1, keepdims=True)
    c1 = jnp.logical_and(biased == m1, valid)
    a1 = jnp.min(jnp.where(c1, lanes, EL), axis=1, keepdims=True)
    oh1 = lanes == a1
    b2 = jnp.where(oh1, NEG, biased)
    m2 = jnp.max(b2, axis=1, keepdims=True)
    c2 = jnp.logical_and(b2 == m2, valid)
    a2 = jnp.min(jnp.where(c2, lanes, EL), axis=1, keepdims=True)
    oh2 = lanes == a2
    oh1f = oh1.astype(jnp.float32)
    oh2f = oh2.astype(jnp.float32)
    l1 = jnp.sum(jnp.where(oh1, lgv, 0.0), axis=1, keepdims=True)
    l2 = jnp.sum(jnp.where(oh2, lgv, 0.0), axis=1, keepdims=True)
    mx = jnp.maximum(l1, l2)
    e1 = jnp.exp(l1 - mx)
    e2 = jnp.exp(l2 - mx)
    s = e1 + e2
    g0_ref[:] = e1 / s
    g1_ref[:] = e2 / s
    S = oh1f + oh2f
    tri = (lax.broadcasted_iota(jnp.int32, (T, T), 0)
           > lax.broadcasted_iota(jnp.int32, (T, T), 1)).astype(jnp.bfloat16)
    EXc = lax.dot_general(tri, S.astype(jnp.bfloat16),
                          (((1,), (0,)), ((), ())),
                          preferred_element_type=jnp.float32)
    rank0 = jnp.sum(oh1f * EXc, axis=1, keepdims=True)
    rank1 = jnp.sum(oh2f * (EXc + oh1f), axis=1, keepdims=True)
    cnt = jnp.sum(S, axis=0, keepdims=True)
    nb = (cnt.astype(jnp.int32) + (TM - 1)) // TM
    tri8 = (lax.broadcasted_iota(jnp.int32, (EL, EL), 0)
            < lax.broadcasted_iota(jnp.int32, (EL, EL), 1)).astype(jnp.float32)
    blk_ex = jnp.dot(nb.astype(jnp.float32), tri8,
                     preferred_element_type=jnp.float32)
    rowoff = TM * blk_ex
    d0_ref[:] = (jnp.sum(oh1f * rowoff, axis=1, keepdims=True)
                 + rank0).astype(jnp.int32)
    d1_ref[:] = (jnp.sum(oh2f * rowoff, axis=1, keepdims=True)
                 + rank1).astype(jnp.int32)
    cnt_ref[:] = cnt.astype(jnp.int32)


def _router(xf, rwt, rb):
    return pl.pallas_call(
        _router_body,
        out_shape=(
            jax.ShapeDtypeStruct((T, 1), jnp.int32),
            jax.ShapeDtypeStruct((T, 1), jnp.int32),
            jax.ShapeDtypeStruct((T, 1), jnp.float32),
            jax.ShapeDtypeStruct((T, 1), jnp.float32),
            jax.ShapeDtypeStruct((1, EL), jnp.int32),
        ),
    )(xf, rwt, rb)


_CH = 64


def _dispatch_body(x_hbm, tok_hbm, dest_hbm, xs_hbm, tokv, destv, rowsv,
                   sem_g, sem_s):
    wid = lax.axis_index("s") * 2 + lax.axis_index("c")
    for c in range(T * K // (NW * _CH)):
        base = wid * (T * K // NW) + c * _CH
        pltpu.sync_copy(tok_hbm.at[pl.ds(base, _CH)], tokv)
        pltpu.sync_copy(dest_hbm.at[pl.ds(base, _CH)], destv)
        pltpu.async_copy(x_hbm.at[tokv], rowsv, sem_g).wait()
        pltpu.async_copy(rowsv, xs_hbm.at[destv], sem_s).wait()


def _dispatch(xf, tok, dest):
    mesh = plsc.VectorSubcoreMesh(core_axis_name="c", subcore_axis_name="s")
    fn = pl.kernel(
        _dispatch_body,
        out_type=jax.ShapeDtypeStruct((NP, D), jnp.float32),
        mesh=mesh,
        scratch_types=[
            pltpu.VMEM((_CH,), jnp.int32),
            pltpu.VMEM((_CH,), jnp.int32),
            pltpu.VMEM((_CH, D), jnp.float32),
            pltpu.SemaphoreType.DMA,
            pltpu.SemaphoreType.DMA,
        ],
    )
    return fn(xf, tok, dest)


def _ffn_block(x, w1b, w3b, w2b):
    h1 = lax.dot_general(x, w1b, (((1,), (1,)), ((), ())),
                         preferred_element_type=jnp.float32)
    h3 = lax.dot_general(x, w3b, (((1,), (1,)), ((), ())),
                         preferred_element_type=jnp.float32)
    hidden = h1 * (h3 * jax.nn.sigmoid(h3))
    return lax.dot_general(hidden, w2b, (((1,), (1,)), ((), ())),
                           preferred_element_type=jnp.float32)


def _expert_body(be_ref, xs_ref, w1_ref, w3_ref, w2_ref, out_ref):
    out_ref[:] = _ffn_block(xs_ref[:], w1_ref[0], w3_ref[0], w2_ref[0])


def _experts(be, xs, w1, w3, w2):
    grid_spec = pltpu.PrefetchScalarGridSpec(
        num_scalar_prefetch=1,
        grid=(NB,),
        in_specs=[
            pl.BlockSpec((TM, D), lambda g, be: (g, 0)),
            pl.BlockSpec((1, FF, D), lambda g, be: (be[g], 0, 0)),
            pl.BlockSpec((1, FF, D), lambda g, be: (be[g], 0, 0)),
            pl.BlockSpec((1, D, FF), lambda g, be: (be[g], 0, 0)),
        ],
        out_specs=pl.BlockSpec((TM, D), lambda g, be: (g, 0)),
    )
    return pl.pallas_call(
        _expert_body,
        grid_spec=grid_spec,
        out_shape=jax.ShapeDtypeStruct((NP, D), jnp.float32),
    )(be, xs, w1, w3, w2)


TMS = 256


def _shared_body(x_ref, sw1_ref, sw3_ref, sw2_ref, out_ref):
    out_ref[:] = _ffn_block(x_ref[:], sw1_ref[:], sw3_ref[:], sw2_ref[:])


def _shared(xf, sw1, sw3, sw2):
    return pl.pallas_call(
        _shared_body,
        grid=(T // TMS,),
        in_specs=[
            pl.BlockSpec((TMS, D), lambda g: (g, 0)),
            pl.BlockSpec((FF, D), lambda g: (0, 0)),
            pl.BlockSpec((FF, D), lambda g: (0, 0)),
            pl.BlockSpec((D, FF), lambda g: (0, 0)),
        ],
        out_specs=pl.BlockSpec((TMS, D), lambda g: (g, 0)),
        out_shape=jax.ShapeDtypeStruct((T, D), jnp.float32),
    )(xf, sw1, sw3, sw2)


_CT = 32


def _combine_body(eo_hbm, sh_hbm, d0_hbm, d1_hbm, g0_hbm, g1_hbm, out_hbm,
                  d0v, d1v, g0v, g1v, av, bv, cv, sem_a, sem_b):
    wid = lax.axis_index("s") * 2 + lax.axis_index("c")
    for c in range(T // (NW * _CT)):
        base = wid * (T // NW) + c * _CT
        pltpu.sync_copy(d0_hbm.at[pl.ds(base, _CT)], d0v)
        pltpu.sync_copy(d1_hbm.at[pl.ds(base, _CT)], d1v)
        pltpu.sync_copy(g0_hbm.at[pl.ds(base, _CT)], g0v)
        pltpu.sync_copy(g1_hbm.at[pl.ds(base, _CT)], g1v)
        pltpu.sync_copy(sh_hbm.at[pl.ds(base, _CT)], cv)
        ca = pltpu.async_copy(eo_hbm.at[d0v], av, sem_a)
        cb = pltpu.async_copy(eo_hbm.at[d1v], bv, sem_b)
        ca.wait()
        cb.wait()

        def tok_body(t, _):
            g0b = g0v[t, :]
            g1b = g1v[t, :]

            def col_body(cc, _):
                off = cc * 16
                a = av[t, pl.ds(off, 16)]
                b = bv[t, pl.ds(off, 16)]
                c0 = cv[t, pl.ds(off, 16)]
                cv[t, pl.ds(off, 16)] = g0b * a + g1b * b + c0
                return 0

            lax.fori_loop(0, D // 16, col_body, 0)
            return 0

        lax.fori_loop(0, _CT, tok_body, 0)
        pltpu.sync_copy(cv, out_hbm.at[pl.ds(base, _CT)])


def _combine(eo, sh, d0, d1, g0, g1):
    mesh = plsc.VectorSubcoreMesh(core_axis_name="c", subcore_axis_name="s")
    fn = pl.kernel(
        _combine_body,
        out_type=jax.ShapeDtypeStruct((T, D), jnp.float32),
        mesh=mesh,
        scratch_types=[
            pltpu.VMEM((_CT,), jnp.int32),
            pltpu.VMEM((_CT,), jnp.int32),
            pltpu.VMEM((_CT, 16), jnp.float32),
            pltpu.VMEM((_CT, 16), jnp.float32),
            pltpu.VMEM((_CT, D), jnp.float32),
            pltpu.VMEM((_CT, D), jnp.float32),
            pltpu.VMEM((_CT, D), jnp.float32),
            pltpu.SemaphoreType.DMA,
            pltpu.SemaphoreType.DMA,
        ],
    )
    return fn(eo, sh, d0, d1, g0, g1)


def kernel(x, router_w, router_b, w1, w3, w2, sw1, sw3, sw2):
    Bsz, Sl, Dm = x.shape
    xf = x.reshape(-1, Dm)
    rwt = jnp.zeros((D, EL), jnp.float32).at[:, :E].set(router_w.T)
    rb = jnp.zeros((1, EL), jnp.float32).at[0, :E].set(router_b)

    d0, d1, g0, g1, cnt = _router(xf, rwt, rb)
    d0 = d0.reshape(T)
    d1 = d1.reshape(T)
    g0 = g0.reshape(T)
    g1 = g1.reshape(T)

    nb = (cnt[0, :E] + TM - 1) // TM
    inc = jnp.cumsum(nb)
    g_ids = jnp.arange(NB, dtype=jnp.int32)[:, None]
    be = jnp.minimum(jnp.sum((inc[None, :] <= g_ids).astype(jnp.int32), axis=1),
                     E - 1)

    dest = jnp.stack([d0, d1], axis=1).reshape(T * K)
    tok = jnp.repeat(jnp.arange(T, dtype=jnp.int32), K)

    xs = _dispatch(xf, tok, dest)
    eo = _experts(be, xs, w1, w3, w2)
    sh = _shared(xf, sw1, sw3, sw2)
    g0r = jnp.broadcast_to(g0[:, None], (T, 16))
    g1r = jnp.broadcast_to(g1[:, None], (T, 16))
    out = _combine(eo, sh, d0, d1, g0r, g1r)
    return out.reshape(Bsz, Sl, Dm)
